# baseline (speedup 1.0000x reference)
import functools

import jax
import jax.numpy as jnp
from jax import lax
from jax.experimental import pallas as pl
from jax.experimental.pallas import tpu as pltpu
from jax.experimental.pallas import tpu_sc as plsc

NC = 2
NS = 16
L = 16
NW = NC * NS

B = 16384
D = 64
BPW = B // NW
C = 128
NCHUNK = BPW // C
NG = C // L


def _rsqrt(x):
    i = lax.bitcast_convert_type(x, jnp.int32)
    i = jnp.int32(0x5F3759DF) - lax.shift_right_arithmetic(i, 1)
    y = lax.bitcast_convert_type(i, jnp.float32)
    for _ in range(3):
        y = y * (1.5 - 0.5 * x * y * y)
    return y


def _sqrt(x):
    m = jnp.maximum(x, 1e-30)
    s = m * _rsqrt(m)
    return jnp.where(x > 0.0, s, 0.0)


def _inv_norm(ss):
    return 1.0 / jnp.maximum(_sqrt(ss), 1e-12)


def _body(h_hbm, t_hbm, nh_hbm, nt_hbm, rel_hbm, ent_hbm, remb_hbm, nv_hbm,
          gold_hbm, neg_hbm,
          hix, tix, nhix, ntix, rix,
          bufs0, bufs1, gold_v, neg_v, sem0, sem1):
    wid = lax.axis_index("s") * NC + lax.axis_index("c")
    base = wid * BPW

    pltpu.sync_copy(h_hbm.at[pl.ds(base, BPW)], hix)
    pltpu.sync_copy(t_hbm.at[pl.ds(base, BPW)], tix)
    pltpu.sync_copy(nh_hbm.at[pl.ds(base, BPW)], nhix)
    pltpu.sync_copy(nt_hbm.at[pl.ds(base, BPW)], ntix)
    pltpu.sync_copy(rel_hbm.at[pl.ds(base, BPW)], rix)

    bufs = (bufs0, bufs1)
    sems = (sem0, sem1)

    def fire(k, slot):
        off = k * C
        hb, tb, nhb, ntb, nvb, rb = bufs[slot]
        sem = sems[slot]
        return [
            pltpu.async_copy(ent_hbm.at[hix.at[pl.ds(off, C)]], hb, sem),
            pltpu.async_copy(ent_hbm.at[tix.at[pl.ds(off, C)]], tb, sem),
            pltpu.async_copy(ent_hbm.at[nhix.at[pl.ds(off, C)]], nhb, sem),
            pltpu.async_copy(ent_hbm.at[ntix.at[pl.ds(off, C)]], ntb, sem),
            pltpu.async_copy(nv_hbm.at[rix.at[pl.ds(off, C)]], nvb, sem),
            pltpu.async_copy(remb_hbm.at[rix.at[pl.ds(off, C)]], rb, sem),
        ]

    def compute(k, slot):
        hb, tb, nhb, ntb, nvb, rb = bufs[slot]
        zero = jnp.zeros((L,), jnp.float32)

        def group_body(grp, _):
            rows = grp * L + lax.iota(jnp.int32, L)

            def d_body(d, acc):
                cols = jnp.full((L,), d, jnp.int32)
                h = plsc.load_gather(hb, [rows, cols])
                t = plsc.load_gather(tb, [rows, cols])
                a = plsc.load_gather(nhb, [rows, cols])
                b = plsc.load_gather(ntb, [rows, cols])
                n = plsc.load_gather(nvb, [rows, cols])
                r = plsc.load_gather(rb, [rows, cols])
                (shh, stt, sht, shn, stn, shr, strr,
                 saa, sbb, sab, san, sbn, sar, sbr,
                 srr, snn, srn) = acc
                return (shh + h * h, stt + t * t, sht + h * t,
                        shn + h * n, stn + t * n, shr + h * r, strr + t * r,
                        saa + a * a, sbb + b * b, sab + a * b,
                        san + a * n, sbn + b * n, sar + a * r, sbr + b * r,
                        srr + r * r, snn + n * n, srn + r * n)

            (shh, stt, sht, shn, stn, shr, strr,
             saa, sbb, sab, san, sbn, sar, sbr,
             srr, snn, srn) = lax.fori_loop(0, D, d_body, (zero,) * 17)

            ia = _inv_norm(shh)
            ib = _inv_norm(stt)
            p = ia * shn
            q = ib * stn
            c = p - q
            g2 = (ia * ia * shh + ib * ib * stt + srr + c * c * snn
                  + 2.0 * (-(ia * ib) * sht + ia * shr - c * p
                           - ib * strr + c * q - c * srn))
            ja = _inv_norm(saa)
            jb = _inv_norm(sbb)
            pn = ja * san
            qn = jb * sbn
            cn = pn - qn
            n2 = (ja * ja * saa + jb * jb * sbb + srr + cn * cn * snn
                  + 2.0 * (-(ja * jb) * sab + ja * sar - cn * pn
                           - jb * sbr + cn * qn - cn * srn))
            out_off = k * C + grp * L
            gold_v[pl.ds(out_off, L)] = _sqrt(g2)
            neg_v[pl.ds(out_off, L)] = _sqrt(n2)
            return 0

        lax.fori_loop(0, NG, group_body, 0)

    pend = fire(0, 0)
    for k in range(NCHUNK):
        for cp in pend:
            cp.wait()
        if k + 1 < NCHUNK:
            pend = fire(k + 1, (k + 1) % 2)
        compute(k, k % 2)

    pltpu.sync_copy(gold_v, gold_hbm.at[pl.ds(base, BPW)])
    pltpu.sync_copy(neg_v, neg_hbm.at[pl.ds(base, BPW)])


def kernel(heads, tails, negative_heads, negative_tails, relations,
           ent_emb, rel_emb, normal_vectors):
    rowset = [pltpu.VMEM((C, D), jnp.float32) for _ in range(6)]
    run = pl.kernel(
        _body,
        out_type=(
            jax.ShapeDtypeStruct((B,), jnp.float32),
            jax.ShapeDtypeStruct((B,), jnp.float32),
        ),
        mesh=plsc.VectorSubcoreMesh(core_axis_name="c", subcore_axis_name="s",
                                    num_cores=NC, num_subcores=NS),
        compiler_params=pltpu.CompilerParams(
            needs_layout_passes=False, use_tc_tiling_on_sc=False),
        scratch_types=[
            pltpu.VMEM((BPW,), jnp.int32),
            pltpu.VMEM((BPW,), jnp.int32),
            pltpu.VMEM((BPW,), jnp.int32),
            pltpu.VMEM((BPW,), jnp.int32),
            pltpu.VMEM((BPW,), jnp.int32),
            list(rowset),
            [pltpu.VMEM((C, D), jnp.float32) for _ in range(6)],
            pltpu.VMEM((BPW,), jnp.float32),
            pltpu.VMEM((BPW,), jnp.float32),
            pltpu.SemaphoreType.DMA,
            pltpu.SemaphoreType.DMA,
        ],
    )
    return run(heads, tails, negative_heads, negative_tails, relations,
               ent_emb, rel_emb, normal_vectors)


# speedup vs baseline: 1.0223x; 1.0223x over previous
import functools

import jax
import jax.numpy as jnp
from jax import lax
from jax.experimental import pallas as pl
from jax.experimental.pallas import tpu as pltpu
from jax.experimental.pallas import tpu_sc as plsc

NC = 2
NS = 16
L = 16
NW = NC * NS

B = 16384
D = 64
BPW = B // NW
C = 128
NCHUNK = BPW // C
NG = C // L


def _rsqrt(x):
    i = lax.bitcast_convert_type(x, jnp.int32)
    i = jnp.int32(0x5F3759DF) - lax.shift_right_arithmetic(i, 1)
    y = lax.bitcast_convert_type(i, jnp.float32)
    for _ in range(3):
        y = y * (1.5 - 0.5 * x * y * y)
    return y


def _sqrt(x):
    m = jnp.maximum(x, 1e-30)
    s = m * _rsqrt(m)
    return jnp.where(x > 0.0, s, 0.0)


def _inv_norm(ss):
    return 1.0 / jnp.maximum(_sqrt(ss), 1e-12)


def _body(h_hbm, t_hbm, nh_hbm, nt_hbm, rel_hbm, ent_hbm, remb_hbm, nv_hbm,
          gold_hbm, neg_hbm,
          hix, tix, nhix, ntix, rix,
          bufs0, bufs1, gold_v, neg_v, sem0, sem1):
    wid = lax.axis_index("s") * NC + lax.axis_index("c")
    base = wid * BPW

    pltpu.sync_copy(h_hbm.at[pl.ds(base, BPW)], hix)
    pltpu.sync_copy(t_hbm.at[pl.ds(base, BPW)], tix)
    pltpu.sync_copy(nh_hbm.at[pl.ds(base, BPW)], nhix)
    pltpu.sync_copy(nt_hbm.at[pl.ds(base, BPW)], ntix)
    pltpu.sync_copy(rel_hbm.at[pl.ds(base, BPW)], rix)

    bufs = (bufs0, bufs1)
    sems = (sem0, sem1)

    def fire(k, slot):
        off = k * C
        hb, tb, nhb, ntb, nvb, rb = bufs[slot]
        sem = sems[slot]
        return [
            pltpu.async_copy(ent_hbm.at[hix.at[pl.ds(off, C)]], hb, sem),
            pltpu.async_copy(ent_hbm.at[tix.at[pl.ds(off, C)]], tb, sem),
            pltpu.async_copy(ent_hbm.at[nhix.at[pl.ds(off, C)]], nhb, sem),
            pltpu.async_copy(ent_hbm.at[ntix.at[pl.ds(off, C)]], ntb, sem),
            pltpu.async_copy(nv_hbm.at[rix.at[pl.ds(off, C)]], nvb, sem),
            pltpu.async_copy(remb_hbm.at[rix.at[pl.ds(off, C)]], rb, sem),
        ]

    def compute(k, slot):
        hb, tb, nhb, ntb, nvb, rb = bufs[slot]
        zero = jnp.zeros((L,), jnp.float32)

        zrow = jnp.zeros((L,), jnp.int32)

        def group_body(grp, _):
            flat0 = (grp * L + lax.iota(jnp.int32, L)) * D

            def d_body(dblk, carry):
                flat = carry[0]
                acc = carry[1:]
                for _dd in range(4):
                    h = plsc.load_gather(hb, [zrow, flat])
                    t = plsc.load_gather(tb, [zrow, flat])
                    a = plsc.load_gather(nhb, [zrow, flat])
                    b = plsc.load_gather(ntb, [zrow, flat])
                    n = plsc.load_gather(nvb, [zrow, flat])
                    r = plsc.load_gather(rb, [zrow, flat])
                    (shh, stt, sht, shn, stn, shr, strr,
                     saa, sbb, sab, san, sbn, sar, sbr,
                     srr, snn, srn) = acc
                    acc = (shh + h * h, stt + t * t, sht + h * t,
                           shn + h * n, stn + t * n, shr + h * r, strr + t * r,
                           saa + a * a, sbb + b * b, sab + a * b,
                           san + a * n, sbn + b * n, sar + a * r, sbr + b * r,
                           srr + r * r, snn + n * n, srn + r * n)
                    flat = flat + 1
                return (flat,) + acc

            (_, shh, stt, sht, shn, stn, shr, strr,
             saa, sbb, sab, san, sbn, sar, sbr,
             srr, snn, srn) = lax.fori_loop(0, D // 4, d_body,
                                            (flat0,) + (zero,) * 17)

            ia = _inv_norm(shh)
            ib = _inv_norm(stt)
            p = ia * shn
            q = ib * stn
            c = p - q
            g2 = (ia * ia * shh + ib * ib * stt + srr + c * c * snn
                  + 2.0 * (-(ia * ib) * sht + ia * shr - c * p
                           - ib * strr + c * q - c * srn))
            ja = _inv_norm(saa)
            jb = _inv_norm(sbb)
            pn = ja * san
            qn = jb * sbn
            cn = pn - qn
            n2 = (ja * ja * saa + jb * jb * sbb + srr + cn * cn * snn
                  + 2.0 * (-(ja * jb) * sab + ja * sar - cn * pn
                           - jb * sbr + cn * qn - cn * srn))
            out_off = k * C + grp * L
            gold_v[pl.ds(out_off, L)] = _sqrt(g2)
            neg_v[pl.ds(out_off, L)] = _sqrt(n2)
            return 0

        lax.fori_loop(0, NG, group_body, 0)

    pend = fire(0, 0)
    for k in range(NCHUNK):
        for cp in pend:
            cp.wait()
        if k + 1 < NCHUNK:
            pend = fire(k + 1, (k + 1) % 2)
        compute(k, k % 2)

    pltpu.sync_copy(gold_v, gold_hbm.at[pl.ds(base, BPW)])
    pltpu.sync_copy(neg_v, neg_hbm.at[pl.ds(base, BPW)])


def kernel(heads, tails, negative_heads, negative_tails, relations,
           ent_emb, rel_emb, normal_vectors):
    rowset = [pltpu.VMEM((C, D), jnp.float32) for _ in range(6)]
    run = pl.kernel(
        _body,
        out_type=(
            jax.ShapeDtypeStruct((B,), jnp.float32),
            jax.ShapeDtypeStruct((B,), jnp.float32),
        ),
        mesh=plsc.VectorSubcoreMesh(core_axis_name="c", subcore_axis_name="s",
                                    num_cores=NC, num_subcores=NS),
        compiler_params=pltpu.CompilerParams(
            needs_layout_passes=False, use_tc_tiling_on_sc=False),
        scratch_types=[
            pltpu.VMEM((BPW,), jnp.int32),
            pltpu.VMEM((BPW,), jnp.int32),
            pltpu.VMEM((BPW,), jnp.int32),
            pltpu.VMEM((BPW,), jnp.int32),
            pltpu.VMEM((BPW,), jnp.int32),
            list(rowset),
            [pltpu.VMEM((C, D), jnp.float32) for _ in range(6)],
            pltpu.VMEM((BPW,), jnp.float32),
            pltpu.VMEM((BPW,), jnp.float32),
            pltpu.SemaphoreType.DMA,
            pltpu.SemaphoreType.DMA,
        ],
    )
    return run(heads, tails, negative_heads, negative_tails, relations,
               ent_emb, rel_emb, normal_vectors)


# speedup vs baseline: 1.1585x; 1.1332x over previous
import functools

import jax
import jax.numpy as jnp
from jax import lax
from jax.experimental import pallas as pl
from jax.experimental.pallas import tpu as pltpu
from jax.experimental.pallas import tpu_sc as plsc

NC = 2
NS = 16
L = 16
NW = NC * NS

B = 16384
D = 64
BPW = B // NW
C = 128
NCHUNK = BPW // C
NG = C // L


def _rsqrt(x):
    i = lax.bitcast_convert_type(x, jnp.int32)
    i = jnp.int32(0x5F3759DF) - lax.shift_right_arithmetic(i, 1)
    y = lax.bitcast_convert_type(i, jnp.float32)
    for _ in range(3):
        y = y * (1.5 - 0.5 * x * y * y)
    return y


def _sqrt(x):
    m = jnp.maximum(x, 1e-30)
    s = m * _rsqrt(m)
    return jnp.where(x > 0.0, s, 0.0)


def _inv_norm(ss):
    return 1.0 / jnp.maximum(_sqrt(ss), 1e-12)


def _body(h_hbm, t_hbm, nh_hbm, nt_hbm, rel_hbm, ent_hbm, remb_hbm, nv_hbm,
          gold_hbm, neg_hbm,
          hix, tix, nhix, ntix, rix,
          bufs0, bufs1, gold_v, neg_v, sem0, sem1):
    wid = lax.axis_index("s") * NC + lax.axis_index("c")
    base = wid * BPW

    pltpu.sync_copy(h_hbm.at[pl.ds(base, BPW)], hix)
    pltpu.sync_copy(t_hbm.at[pl.ds(base, BPW)], tix)
    pltpu.sync_copy(nh_hbm.at[pl.ds(base, BPW)], nhix)
    pltpu.sync_copy(nt_hbm.at[pl.ds(base, BPW)], ntix)
    pltpu.sync_copy(rel_hbm.at[pl.ds(base, BPW)], rix)

    bufs = (bufs0, bufs1)
    sems = (sem0, sem1)

    def fire(k, slot):
        off = k * C
        hb, tb, nhb, ntb, nvb, rb = bufs[slot]
        sem = sems[slot]
        return [
            pltpu.async_copy(ent_hbm.at[hix.at[pl.ds(off, C)]], hb, sem),
            pltpu.async_copy(ent_hbm.at[tix.at[pl.ds(off, C)]], tb, sem),
            pltpu.async_copy(ent_hbm.at[nhix.at[pl.ds(off, C)]], nhb, sem),
            pltpu.async_copy(ent_hbm.at[ntix.at[pl.ds(off, C)]], ntb, sem),
            pltpu.async_copy(nv_hbm.at[rix.at[pl.ds(off, C)]], nvb, sem),
            pltpu.async_copy(remb_hbm.at[rix.at[pl.ds(off, C)]], rb, sem),
        ]

    def compute(k, slot):
        hb, tb, nhb, ntb, nvb, rb = bufs[slot]
        zero = jnp.zeros((L,), jnp.float32)

        zrow = jnp.zeros((L,), jnp.int32)

        def group_body(grp, _):
            rows = grp * L + lax.iota(jnp.int32, L)
            base = rows * D
            rot0 = jnp.bitwise_and(rows, D - 1)

            def d_body(dblk, carry):
                rot = carry[0]
                acc = carry[1:]
                for _dd in range(4):
                    flat = base + rot
                    h = plsc.load_gather(hb, [zrow, flat])
                    t = plsc.load_gather(tb, [zrow, flat])
                    a = plsc.load_gather(nhb, [zrow, flat])
                    b = plsc.load_gather(ntb, [zrow, flat])
                    n = plsc.load_gather(nvb, [zrow, flat])
                    r = plsc.load_gather(rb, [zrow, flat])
                    (shh, stt, sht, shn, stn, shr, strr,
                     saa, sbb, sab, san, sbn, sar, sbr,
                     srr, snn, srn) = acc
                    acc = (shh + h * h, stt + t * t, sht + h * t,
                           shn + h * n, stn + t * n, shr + h * r, strr + t * r,
                           saa + a * a, sbb + b * b, sab + a * b,
                           san + a * n, sbn + b * n, sar + a * r, sbr + b * r,
                           srr + r * r, snn + n * n, srn + r * n)
                    rot = jnp.bitwise_and(rot + 1, D - 1)
                return (rot,) + acc

            (_, shh, stt, sht, shn, stn, shr, strr,
             saa, sbb, sab, san, sbn, sar, sbr,
             srr, snn, srn) = lax.fori_loop(0, D // 4, d_body,
                                            (rot0,) + (zero,) * 17)

            ia = _inv_norm(shh)
            ib = _inv_norm(stt)
            p = ia * shn
            q = ib * stn
            c = p - q
            g2 = (ia * ia * shh + ib * ib * stt + srr + c * c * snn
                  + 2.0 * (-(ia * ib) * sht + ia * shr - c * p
                           - ib * strr + c * q - c * srn))
            ja = _inv_norm(saa)
            jb = _inv_norm(sbb)
            pn = ja * san
            qn = jb * sbn
            cn = pn - qn
            n2 = (ja * ja * saa + jb * jb * sbb + srr + cn * cn * snn
                  + 2.0 * (-(ja * jb) * sab + ja * sar - cn * pn
                           - jb * sbr + cn * qn - cn * srn))
            out_off = k * C + grp * L
            gold_v[pl.ds(out_off, L)] = _sqrt(g2)
            neg_v[pl.ds(out_off, L)] = _sqrt(n2)
            return 0

        lax.fori_loop(0, NG, group_body, 0)

    pend = fire(0, 0)
    for k in range(NCHUNK):
        for cp in pend:
            cp.wait()
        if k + 1 < NCHUNK:
            pend = fire(k + 1, (k + 1) % 2)
        compute(k, k % 2)

    pltpu.sync_copy(gold_v, gold_hbm.at[pl.ds(base, BPW)])
    pltpu.sync_copy(neg_v, neg_hbm.at[pl.ds(base, BPW)])


def kernel(heads, tails, negative_heads, negative_tails, relations,
           ent_emb, rel_emb, normal_vectors):
    rowset = [pltpu.VMEM((C, D), jnp.float32) for _ in range(6)]
    run = pl.kernel(
        _body,
        out_type=(
            jax.ShapeDtypeStruct((B,), jnp.float32),
            jax.ShapeDtypeStruct((B,), jnp.float32),
        ),
        mesh=plsc.VectorSubcoreMesh(core_axis_name="c", subcore_axis_name="s",
                                    num_cores=NC, num_subcores=NS),
        compiler_params=pltpu.CompilerParams(
            needs_layout_passes=False, use_tc_tiling_on_sc=False),
        scratch_types=[
            pltpu.VMEM((BPW,), jnp.int32),
            pltpu.VMEM((BPW,), jnp.int32),
            pltpu.VMEM((BPW,), jnp.int32),
            pltpu.VMEM((BPW,), jnp.int32),
            pltpu.VMEM((BPW,), jnp.int32),
            list(rowset),
            [pltpu.VMEM((C, D), jnp.float32) for _ in range(6)],
            pltpu.VMEM((BPW,), jnp.float32),
            pltpu.VMEM((BPW,), jnp.float32),
            pltpu.SemaphoreType.DMA,
            pltpu.SemaphoreType.DMA,
        ],
    )
    return run(heads, tails, negative_heads, negative_tails, relations,
               ent_emb, rel_emb, normal_vectors)
